# baseline (speedup 1.0000x reference)
import functools
import math

import jax
import jax.numpy as jnp
from jax import lax
from jax.experimental import pallas as pl
from jax.experimental.pallas import tpu as pltpu
from jax.experimental.pallas import tpu_sc as plsc

B, N, DIM, DH, H, M_D, FF, KN = 4, 512, 128, 64, 4, 16, 4, 32
BN = B * N
INNER = H * DH


def _stage_a_body(feats_ref, w_ref, cq_ref, ck_ref, q_ref, kv_ref, d2_ref):
    f = feats_ref[0]
    qkv = jnp.dot(f, w_ref[...], preferred_element_type=jnp.float32,
                  precision=lax.Precision.DEFAULT)
    q_ref[0] = qkv[:, :INNER]
    kv_ref[0] = qkv[:, INNER:]
    cq = cq_ref[0]
    ck = ck_ref[0]
    d0 = cq[:, 0:1] - ck[0:1, :]
    d1 = cq[:, 1:2] - ck[1:2, :]
    d2c = cq[:, 2:3] - ck[2:3, :]
    d2_ref[0] = d0 * d0 + d1 * d1 + d2c * d2c


def _stage_a(feats, W_qkv, coors_q, coors_kT):
    return pl.pallas_call(
        _stage_a_body,
        grid=(B,),
        in_specs=[
            pl.BlockSpec((1, N, DIM), lambda i: (i, 0, 0)),
            pl.BlockSpec((DIM, 3 * INNER), lambda i: (0, 0)),
            pl.BlockSpec((1, N, 16), lambda i: (i, 0, 0)),
            pl.BlockSpec((1, 16, N), lambda i: (i, 0, 0)),
        ],
        out_specs=[
            pl.BlockSpec((1, N, INNER), lambda i: (i, 0, 0)),
            pl.BlockSpec((1, N, 2 * INNER), lambda i: (i, 0, 0)),
            pl.BlockSpec((1, N, N), lambda i: (i, 0, 0)),
        ],
        out_shape=[
            jax.ShapeDtypeStruct((B, N, INNER), jnp.float32),
            jax.ShapeDtypeStruct((B, N, 2 * INNER), jnp.float32),
            jax.ShapeDtypeStruct((B, N, N), jnp.float32),
        ],
    )(feats, W_qkv, coors_q, coors_kT)


def _merge16(ak, av, bk, bv):
    rbk = lax.rev(bk, (0,))
    rbv = lax.rev(bv, (0,))
    m = ak <= rbk
    lok = jnp.where(m, ak, rbk)
    lov = jnp.where(m, av, rbv)
    hik = jnp.where(m, rbk, ak)
    hiv = jnp.where(m, rbv, av)
    lok, lov = plsc.sort_key_val(lok, lov)
    hik, hiv = plsc.sort_key_val(hik, hiv)
    return lok, lov, hik, hiv


def _sc_select_gather(d2, kv, ctab, rows):
    info = plsc.get_sparse_core_info()
    nw = info.num_cores * info.num_subcores
    q_per_w = rows // nw
    mesh = plsc.VectorSubcoreMesh(core_axis_name="c", subcore_axis_name="s")

    @functools.partial(
        pl.kernel, mesh=mesh,
        compiler_params=pltpu.CompilerParams(needs_layout_passes=False),
        out_type=[
            jax.ShapeDtypeStruct((rows * KN, 2 * INNER), jnp.float32),
            jax.ShapeDtypeStruct((rows * KN, 16), jnp.float32),
        ],
        scratch_types=[
            pltpu.VMEM((8, N), jnp.float32),
            pltpu.VMEM((2, KN), jnp.int32),
            pltpu.VMEM((2, KN, 2 * INNER), jnp.float32),
            pltpu.VMEM((2, KN, 16), jnp.float32),
            pltpu.VMEM((16, N), jnp.float32),
            pltpu.SemaphoreType.DMA,
            pltpu.SemaphoreType.DMA,
            pltpu.SemaphoreType.DMA,
            pltpu.SemaphoreType.DMA,
            pltpu.SemaphoreType.DMA,
        ],
    )
    def sc_kernel(d2_hbm, kv_hbm, ct_hbm, kvnb_hbm, cnb_hbm,
                  rows_v, idx_v, kvrows_v, crows_v, ctab_v,
                  sem_g, sem_k0, sem_k1, sem_c0, sem_c1):
        wid = lax.axis_index("s") * info.num_cores + lax.axis_index("c")
        base_q = wid * q_per_w
        b = base_q // N
        pltpu.sync_copy(ct_hbm, ctab_v)
        lane16 = lax.iota(jnp.int32, 16)
        zero16 = jnp.zeros((16,), jnp.float32)
        for bu in range(2):
            for r in range(KN):
                crows_v[bu, r] = zero16

        def wait_copyout(bu, sk, sc_):
            pltpu.make_async_copy(
                kvrows_v.at[bu], kvnb_hbm.at[pl.ds(0, KN)], sk).wait()
            pltpu.make_async_copy(
                crows_v.at[bu], cnb_hbm.at[pl.ds(0, KN)], sc_).wait()

        def wait_gather(bu):
            pltpu.make_async_copy(
                kv_hbm.at[idx_v.at[bu]], kvrows_v.at[bu], sem_g).wait()

        def copyout(bu, qg, sk, sc_):
            pltpu.async_copy(kvrows_v.at[bu],
                             kvnb_hbm.at[pl.ds(qg * KN, KN)], sk)
            pltpu.async_copy(crows_v.at[bu],
                             cnb_hbm.at[pl.ds(qg * KN, KN)], sc_)

        def one_query(qi, carry):
            qg = base_q + qi

            @pl.when(qi % 8 == 0)
            def _():
                qg8 = pl.multiple_of(base_q + qi, 8)
                pltpu.sync_copy(d2_hbm.at[pl.ds(qg8, 8)], rows_v)

            ri = qi % 8
            lane = lane16

            def chunk(c):
                ck = rows_v[ri, pl.ds(c * 16, 16)]
                cv = lane + c * 16
                return plsc.sort_key_val(ck, cv)

            b0k, b0v = chunk(0)
            b1k, b1v = chunk(1)
            b0k, b0v, b1k, b1v = _merge16(b0k, b0v, b1k, b1v)

            for c in range(2, N // 16):
                ck, cv = chunk(c)
                rck = lax.rev(ck, (0,))
                rcv = lax.rev(cv, (0,))
                m = b1k <= rck
                lok = jnp.where(m, b1k, rck)
                lov = jnp.where(m, b1v, rcv)
                lok, lov = plsc.sort_key_val(lok, lov)
                b0k, b0v, b1k, b1v = _merge16(b0k, b0v, lok, lov)

            bu = qi % 2
            @pl.when(jnp.logical_and(qi >= 2, bu == 0))
            def _():
                wait_copyout(0, sem_k0, sem_c0)

            @pl.when(jnp.logical_and(qi >= 2, bu == 1))
            def _():
                wait_copyout(1, sem_k1, sem_c1)

            batch_base = b * N
            idx_v[bu, pl.ds(0, 16)] = b0v + batch_base
            idx_v[bu, pl.ds(16, 16)] = b1v + batch_base
            pltpu.async_copy(kv_hbm.at[idx_v.at[bu]], kvrows_v.at[bu], sem_g)
            for half, loc in ((0, b0v), (1, b1v)):
                rows = lane + half * 16
                for c in range(3):
                    g = plsc.load_gather(ctab_v, [lane * 0 + (b * 3 + c), loc])
                    plsc.store_scatter(crows_v.at[bu], [rows, lane * 0 + c], g)

            @pl.when(jnp.logical_and(qi >= 1, bu == 1))
            def _():
                wait_gather(0)
                copyout(0, qg - 1, sem_k0, sem_c0)

            @pl.when(jnp.logical_and(qi >= 1, bu == 0))
            def _():
                wait_gather(1)
                copyout(1, qg - 1, sem_k1, sem_c1)

            return carry

        lax.fori_loop(0, q_per_w, one_query, 0)
        last = q_per_w - 1
        lb = last % 2
        wait_gather(lb)
        copyout(lb, base_q + last, (sem_k0, sem_k1)[lb], (sem_c0, sem_c1)[lb])
        wait_copyout(0, sem_k0, sem_c0)
        wait_copyout(1, sem_k1, sem_c1)

    return sc_kernel(d2, kv, ctab)


TQ = 128
TE = TQ * KN


def _rep_q(x, cols):
    return jnp.broadcast_to(x[:, None, :], (TQ, KN, cols)).reshape(TE, cols)


def _stage_c_body(q_ref, kvnb_ref, cnb_ref, cq_ref, sv_ref, pv_ref,
                  wp1_ref, bp1_ref, wp2_ref, bp2_ref,
                  w1a_ref, w1b_ref, be1_ref, we2_ref, be2_ref,
                  wac_ref, bac_ref, wac2_ref, bac2_ref,
                  wout_ref, bout_ref, wco_ref,
                  out_ref, cout_ref):
    dot = functools.partial(jnp.dot, preferred_element_type=jnp.float32,
                            precision=lax.Precision.DEFAULT)
    cq = _rep_q(cq_ref[...], 16)
    relc = cq - cnb_ref[...]
    relc_t = jnp.transpose(relc)
    d2_t = (relc_t[0:1] * relc_t[0:1] + relc_t[1:2] * relc_t[1:2]
            + relc_t[2:3] * relc_t[2:3])
    dist_t = jnp.sqrt(jnp.maximum(d2_t, 1e-12))
    ang_t = dist_t * jnp.transpose(sv_ref[...]) + jnp.transpose(pv_ref[...])
    rd_t = jnp.sin(ang_t)
    rows16 = lax.broadcasted_iota(jnp.int32, (16, 1), 0)
    rd_t = jnp.where(rows16 == 8, dist_t, rd_t)
    rd = jnp.transpose(rd_t)
    pos = dot(jax.nn.relu(dot(rd, wp1_ref[...]) + bp1_ref[...]),
              wp2_ref[...]) + bp2_ref[...]
    hpos = dot(pos, w1b_ref[...]) + be1_ref[...]

    ms = []
    for h in range(H):
        qh = _rep_q(q_ref[:, h * DH:(h + 1) * DH], DH)
        kh = kvnb_ref[:, h * DH:(h + 1) * DH]
        qk = qh * kh
        h1 = jax.nn.relu(dot(qk, w1a_ref[...]) + hpos)
        ms.append(jax.nn.relu(dot(h1, we2_ref[...]) + be2_ref[...]))
    m_all = jnp.concatenate(ms, axis=1)
    hh = jax.nn.relu(dot(m_all, wac_ref[...]) + bac_ref[...])
    sc = dot(hh, wac2_ref[...]) + bac2_ref[...]
    ea = jnp.exp(sc)
    den = jnp.sum(ea.reshape(TQ, KN, 8), axis=1, keepdims=True)
    attn_all = (ea.reshape(TQ, KN, 8) / den).reshape(TE, 8)
    outs = []
    cwc = jnp.zeros((TE, 1), jnp.float32)
    for h in range(H):
        vh = kvnb_ref[:, INNER + h * DH:INNER + (h + 1) * DH] + pos
        cw = sc[:, 2 * h + 1:2 * h + 2]
        attn = attn_all[:, 2 * h:2 * h + 1]
        outs.append(jnp.sum((attn * vh).reshape(TQ, KN, DH), axis=1))
        cwc = cwc + cw * wco_ref[0:1, h:h + 1]
    cr = jnp.sum((cwc * relc).reshape(TQ, KN, 16), axis=1)
    cout_ref[...] = cr
    o = jnp.concatenate(outs, axis=1)
    out_ref[...] = dot(o, wout_ref[...]) + bout_ref[...]


def _stage_c(q, kv_nb, c_nb, coorsP, consts, rows):
    (sv, pv, Wp1p, bp1, Wp2, bp2, W1a, W1b, be1, We2, be2,
     Wac, bac, Wac2, bac2, W_out, b_out, WcoP) = consts
    n_tiles = rows // TQ
    wspec = lambda shape: pl.BlockSpec(shape, lambda i: tuple(0 for _ in shape))
    return pl.pallas_call(
        _stage_c_body,
        grid=(n_tiles,),
        in_specs=[
            pl.BlockSpec((TQ, INNER), lambda i: (i, 0)),
            pl.BlockSpec((TE, 2 * INNER), lambda i: (i, 0)),
            pl.BlockSpec((TE, 16), lambda i: (i, 0)),
            pl.BlockSpec((TQ, 16), lambda i: (i, 0)),
            wspec((1, 16)), wspec((1, 16)),
            wspec((16, 2 * DH)), wspec((1, 2 * DH)),
            wspec((2 * DH, DH)), wspec((1, DH)),
            wspec((DH, 4 * DH)), wspec((DH, 4 * DH)), wspec((1, 4 * DH)),
            wspec((4 * DH, M_D)), wspec((1, M_D)),
            wspec((H * M_D, H * 8 * M_D)), wspec((1, H * 8 * M_D)),
            wspec((H * 8 * M_D, 8)), wspec((1, 8)),
            wspec((INNER, DIM)), wspec((1, DIM)),
            wspec((1, 8)),
        ],
        out_specs=[
            pl.BlockSpec((TQ, DIM), lambda i: (i, 0)),
            pl.BlockSpec((TQ, 16), lambda i: (i, 0)),
        ],
        out_shape=[
            jax.ShapeDtypeStruct((rows, DIM), jnp.float32),
            jax.ShapeDtypeStruct((rows, 16), jnp.float32),
        ],
    )(q, kv_nb, c_nb, coorsP, sv, pv, Wp1p, bp1, Wp2, bp2, W1a, W1b, be1,
      We2, be2, Wac, bac, Wac2, bac2, W_out, b_out, WcoP)


def _prep_consts(Wp1, bp1, Wp2, bp2, We1, be1, We2, be2,
                 Wa1, ba1, Wa2, ba2, Wc1, bc1, Wc2, bc2, Wco, W_out, b_out):
    f32 = jnp.float32
    scales = 2.0 ** jnp.arange(FF, dtype=f32)
    sv = jnp.zeros((1, 16), f32)
    sv = sv.at[0, :FF].set(1.0 / scales).at[0, FF:2 * FF].set(1.0 / scales)
    pv = jnp.zeros((1, 16), f32).at[0, FF:2 * FF].set(0.5 * math.pi)
    Wp1p = jnp.zeros((16, 2 * DH), f32).at[: FF * 2 + 1].set(Wp1)
    W1a, W1b = We1[:DH], We1[DH:]
    Wac1h = jnp.concatenate([Wa1, Wc1], axis=1)
    bac1h = jnp.concatenate([ba1, bc1], axis=0)
    Wac = jnp.zeros((H * M_D, H * 8 * M_D), f32)
    for h in range(H):
        Wac = Wac.at[h * M_D:(h + 1) * M_D,
                     h * 8 * M_D:(h + 1) * 8 * M_D].set(Wac1h)
    bac = jnp.tile(bac1h, (H,))[None]
    Wac2 = jnp.zeros((H * 8 * M_D, 8), f32)
    bac2 = jnp.zeros((1, 8), f32)
    for h in range(H):
        r0 = h * 8 * M_D
        Wac2 = Wac2.at[r0:r0 + 4 * M_D, 2 * h:2 * h + 1].set(Wa2)
        Wac2 = Wac2.at[r0 + 4 * M_D:r0 + 8 * M_D,
                       2 * h + 1:2 * h + 2].set(Wc2)
        bac2 = bac2.at[0, 2 * h].set(ba2[0]).at[0, 2 * h + 1].set(bc2[0])
    WcoP = jnp.zeros((1, 8), f32).at[0, :H].set(Wco[:, 0])
    return (sv, pv, Wp1p, bp1[None], Wp2, bp2[None], W1a, W1b, be1[None],
            We2, be2[None], Wac, bac, Wac2, bac2, W_out, b_out[None], WcoP)


def kernel(feats, coors, W_qkv, W_out, b_out, Wp1, bp1, Wp2, bp2,
           We1, be1, We2, be2, Wa1, ba1, Wa2, ba2, Wc1, bc1, Wc2, bc2, Wco):
    coorsP = jnp.pad(coors, ((0, 0), (0, 0), (0, 13)))
    coors_kT = jnp.swapaxes(coorsP, 1, 2)
    q, kv, d2 = _stage_a(feats, W_qkv, coorsP, coors_kT)
    consts = _prep_consts(Wp1, bp1, Wp2, bp2, We1, be1, We2, be2,
                          Wa1, ba1, Wa2, ba2, Wc1, bc1, Wc2, bc2, Wco,
                          W_out, b_out)
    nbs = [_sc_select_gather(d2[b], kv[b], coors_kT[b], N) for b in range(B)]
    outs, couts = [], []
    for b in range(B):
        kv_nb, c_nb = nbs[b]
        o, co = _stage_c(q[b], kv_nb, c_nb, coorsP[b], consts, N)
        outs.append(o)
        couts.append(co[:, :3])
    return jnp.stack(outs), jnp.stack(couts)


# speedup vs baseline: 1.0630x; 1.0630x over previous
import functools
import math

import jax
import jax.numpy as jnp
from jax import lax
from jax.experimental import pallas as pl
from jax.experimental.pallas import tpu as pltpu
from jax.experimental.pallas import tpu_sc as plsc

B, N, DIM, DH, H, M_D, FF, KN = 4, 512, 128, 64, 4, 16, 4, 32
BN = B * N
INNER = H * DH


def _stage_a_body(feats_ref, w_ref, cq_ref, ck_ref, q_ref, kv_ref, d2_ref):
    f = feats_ref[0]
    qkv = jnp.dot(f, w_ref[...], preferred_element_type=jnp.float32,
                  precision=lax.Precision.DEFAULT)
    q_ref[0] = qkv[:, :INNER]
    kv_ref[0] = qkv[:, INNER:]
    cq = cq_ref[0]
    ck = ck_ref[0]
    d0 = cq[:, 0:1] - ck[0:1, :]
    d1 = cq[:, 1:2] - ck[1:2, :]
    d2c = cq[:, 2:3] - ck[2:3, :]
    d2_ref[0] = d0 * d0 + d1 * d1 + d2c * d2c


def _stage_a(feats, W_qkv, coors_q, coors_kT):
    return pl.pallas_call(
        _stage_a_body,
        grid=(B,),
        in_specs=[
            pl.BlockSpec((1, N, DIM), lambda i: (i, 0, 0)),
            pl.BlockSpec((DIM, 3 * INNER), lambda i: (0, 0)),
            pl.BlockSpec((1, N, 16), lambda i: (i, 0, 0)),
            pl.BlockSpec((1, 16, N), lambda i: (i, 0, 0)),
        ],
        out_specs=[
            pl.BlockSpec((1, N, INNER), lambda i: (i, 0, 0)),
            pl.BlockSpec((1, N, 2 * INNER), lambda i: (i, 0, 0)),
            pl.BlockSpec((1, N, N), lambda i: (i, 0, 0)),
        ],
        out_shape=[
            jax.ShapeDtypeStruct((B, N, INNER), jnp.float32),
            jax.ShapeDtypeStruct((B, N, 2 * INNER), jnp.float32),
            jax.ShapeDtypeStruct((B, N, N), jnp.float32),
        ],
    )(feats, W_qkv, coors_q, coors_kT)


def _merge16(ak, av, bk, bv):
    rbk = lax.rev(bk, (0,))
    rbv = lax.rev(bv, (0,))
    m = ak <= rbk
    lok = jnp.where(m, ak, rbk)
    lov = jnp.where(m, av, rbv)
    hik = jnp.where(m, rbk, ak)
    hiv = jnp.where(m, rbv, av)
    lok, lov = plsc.sort_key_val(lok, lov)
    hik, hiv = plsc.sort_key_val(hik, hiv)
    return lok, lov, hik, hiv


def _sc_select_gather(d2, kv, ctab, rows):
    info = plsc.get_sparse_core_info()
    nw = info.num_cores * info.num_subcores
    q_per_w = rows // nw
    mesh = plsc.VectorSubcoreMesh(core_axis_name="c", subcore_axis_name="s")

    @functools.partial(
        pl.kernel, mesh=mesh,
        compiler_params=pltpu.CompilerParams(needs_layout_passes=False),
        out_type=[
            jax.ShapeDtypeStruct((rows * KN, 2 * INNER), jnp.float32),
            jax.ShapeDtypeStruct((rows * KN, 16), jnp.float32),
        ],
        scratch_types=[
            pltpu.VMEM((8, N), jnp.float32),
            pltpu.VMEM((2, KN), jnp.int32),
            pltpu.VMEM((2, KN, 2 * INNER), jnp.float32),
            pltpu.VMEM((2, KN, 16), jnp.float32),
            pltpu.VMEM((16, N), jnp.float32),
            pltpu.SemaphoreType.DMA,
            pltpu.SemaphoreType.DMA,
            pltpu.SemaphoreType.DMA,
            pltpu.SemaphoreType.DMA,
            pltpu.SemaphoreType.DMA,
        ],
    )
    def sc_kernel(d2_hbm, kv_hbm, ct_hbm, kvnb_hbm, cnb_hbm,
                  rows_v, idx_v, kvrows_v, crows_v, ctab_v,
                  sem_g, sem_k0, sem_k1, sem_c0, sem_c1):
        wid = lax.axis_index("s") * info.num_cores + lax.axis_index("c")
        base_q = wid * q_per_w
        b = base_q // N
        pltpu.sync_copy(ct_hbm, ctab_v)
        lane16 = lax.iota(jnp.int32, 16)
        zero16 = jnp.zeros((16,), jnp.float32)
        for bu in range(2):
            for r in range(KN):
                crows_v[bu, r] = zero16

        def wait_copyout(bu, sk, sc_):
            pltpu.make_async_copy(
                kvrows_v.at[bu], kvnb_hbm.at[pl.ds(0, KN)], sk).wait()
            pltpu.make_async_copy(
                crows_v.at[bu], cnb_hbm.at[pl.ds(0, KN)], sc_).wait()

        def wait_gather(bu):
            pltpu.make_async_copy(
                kv_hbm.at[idx_v.at[bu]], kvrows_v.at[bu], sem_g).wait()

        def copyout(bu, qg, sk, sc_):
            pltpu.async_copy(kvrows_v.at[bu],
                             kvnb_hbm.at[pl.ds(qg * KN, KN)], sk)
            pltpu.async_copy(crows_v.at[bu],
                             cnb_hbm.at[pl.ds(qg * KN, KN)], sc_)

        def one_query(qi, carry):
            qg = base_q + qi

            @pl.when(qi % 8 == 0)
            def _():
                qg8 = pl.multiple_of(base_q + qi, 8)
                pltpu.sync_copy(d2_hbm.at[pl.ds(qg8, 8)], rows_v)

            ri = qi % 8
            lane = lane16

            def chunk(c):
                ck = rows_v[ri, pl.ds(c * 16, 16)]
                cv = lane + c * 16
                return plsc.sort_key_val(ck, cv)

            b0k, b0v = chunk(0)
            b1k, b1v = chunk(1)
            b0k, b0v, b1k, b1v = _merge16(b0k, b0v, b1k, b1v)

            for c in range(2, N // 16):
                ck, cv = chunk(c)
                rck = lax.rev(ck, (0,))
                rcv = lax.rev(cv, (0,))
                m = b1k <= rck
                lok = jnp.where(m, b1k, rck)
                lov = jnp.where(m, b1v, rcv)
                lok, lov = plsc.sort_key_val(lok, lov)
                b0k, b0v, b1k, b1v = _merge16(b0k, b0v, lok, lov)

            bu = qi % 2
            @pl.when(jnp.logical_and(qi >= 2, bu == 0))
            def _():
                wait_copyout(0, sem_k0, sem_c0)

            @pl.when(jnp.logical_and(qi >= 2, bu == 1))
            def _():
                wait_copyout(1, sem_k1, sem_c1)

            batch_base = b * N
            idx_v[bu, pl.ds(0, 16)] = b0v + batch_base
            idx_v[bu, pl.ds(16, 16)] = b1v + batch_base
            cp1 = pltpu.async_copy(kv_hbm.at[idx_v.at[bu]], kvrows_v.at[bu],
                                   sem_g)
            for half, loc in ((0, b0v), (1, b1v)):
                rows = lane + half * 16
                for c in range(3):
                    g = plsc.load_gather(ctab_v, [lane * 0 + (b * 3 + c), loc])
                    plsc.store_scatter(crows_v.at[bu], [rows, lane * 0 + c], g)
            cp1.wait()

            @pl.when(bu == 0)
            def _():
                copyout(0, qg, sem_k0, sem_c0)

            @pl.when(bu == 1)
            def _():
                copyout(1, qg, sem_k1, sem_c1)

            return carry

        lax.fori_loop(0, q_per_w, one_query, 0)
        wait_copyout(0, sem_k0, sem_c0)
        wait_copyout(1, sem_k1, sem_c1)

    return sc_kernel(d2, kv, ctab)


TQ = 128
TE = TQ * KN


def _rep_q(x, cols):
    return jnp.broadcast_to(x[:, None, :], (TQ, KN, cols)).reshape(TE, cols)


def _stage_c_body(q_ref, kvnb_ref, cnb_ref, cq_ref, sv_ref, pv_ref,
                  wp1_ref, bp1_ref, wp2_ref, bp2_ref,
                  w1a_ref, w1b_ref, be1_ref, we2_ref, be2_ref,
                  wac_ref, bac_ref, wac2_ref, bac2_ref,
                  wout_ref, bout_ref, wco_ref,
                  out_ref, cout_ref):
    dot = functools.partial(jnp.dot, preferred_element_type=jnp.float32,
                            precision=lax.Precision.DEFAULT)
    cq = _rep_q(cq_ref[...], 16)
    relc = cq - cnb_ref[...]
    relc_t = jnp.transpose(relc)
    d2_t = (relc_t[0:1] * relc_t[0:1] + relc_t[1:2] * relc_t[1:2]
            + relc_t[2:3] * relc_t[2:3])
    dist_t = jnp.sqrt(jnp.maximum(d2_t, 1e-12))
    ang_t = dist_t * jnp.transpose(sv_ref[...]) + jnp.transpose(pv_ref[...])
    rd_t = jnp.sin(ang_t)
    rows16 = lax.broadcasted_iota(jnp.int32, (16, 1), 0)
    rd_t = jnp.where(rows16 == 8, dist_t, rd_t)
    rd = jnp.transpose(rd_t)
    pos = dot(jax.nn.relu(dot(rd, wp1_ref[...]) + bp1_ref[...]),
              wp2_ref[...]) + bp2_ref[...]
    hpos = dot(pos, w1b_ref[...]) + be1_ref[...]

    ms = []
    for h in range(H):
        qh = _rep_q(q_ref[:, h * DH:(h + 1) * DH], DH)
        kh = kvnb_ref[:, h * DH:(h + 1) * DH]
        qk = qh * kh
        h1 = jax.nn.relu(dot(qk, w1a_ref[...]) + hpos)
        ms.append(jax.nn.relu(dot(h1, we2_ref[...]) + be2_ref[...]))
    m_all = jnp.concatenate(ms, axis=1)
    hh = jax.nn.relu(dot(m_all, wac_ref[...]) + bac_ref[...])
    sc = dot(hh, wac2_ref[...]) + bac2_ref[...]
    ea = jnp.exp(sc)
    den = jnp.sum(ea.reshape(TQ, KN, 8), axis=1, keepdims=True)
    attn_all = (ea.reshape(TQ, KN, 8) / den).reshape(TE, 8)
    outs = []
    cwc = jnp.zeros((TE, 1), jnp.float32)
    for h in range(H):
        vh = kvnb_ref[:, INNER + h * DH:INNER + (h + 1) * DH] + pos
        cw = sc[:, 2 * h + 1:2 * h + 2]
        attn = attn_all[:, 2 * h:2 * h + 1]
        outs.append(jnp.sum((attn * vh).reshape(TQ, KN, DH), axis=1))
        cwc = cwc + cw * wco_ref[0:1, h:h + 1]
    cr = jnp.sum((cwc * relc).reshape(TQ, KN, 16), axis=1)
    cout_ref[...] = cr
    o = jnp.concatenate(outs, axis=1)
    out_ref[...] = dot(o, wout_ref[...]) + bout_ref[...]


def _stage_c(q, kv_nb, c_nb, coorsP, consts, rows):
    (sv, pv, Wp1p, bp1, Wp2, bp2, W1a, W1b, be1, We2, be2,
     Wac, bac, Wac2, bac2, W_out, b_out, WcoP) = consts
    n_tiles = rows // TQ
    wspec = lambda shape: pl.BlockSpec(shape, lambda i: tuple(0 for _ in shape))
    return pl.pallas_call(
        _stage_c_body,
        grid=(n_tiles,),
        in_specs=[
            pl.BlockSpec((TQ, INNER), lambda i: (i, 0)),
            pl.BlockSpec((TE, 2 * INNER), lambda i: (i, 0)),
            pl.BlockSpec((TE, 16), lambda i: (i, 0)),
            pl.BlockSpec((TQ, 16), lambda i: (i, 0)),
            wspec((1, 16)), wspec((1, 16)),
            wspec((16, 2 * DH)), wspec((1, 2 * DH)),
            wspec((2 * DH, DH)), wspec((1, DH)),
            wspec((DH, 4 * DH)), wspec((DH, 4 * DH)), wspec((1, 4 * DH)),
            wspec((4 * DH, M_D)), wspec((1, M_D)),
            wspec((H * M_D, H * 8 * M_D)), wspec((1, H * 8 * M_D)),
            wspec((H * 8 * M_D, 8)), wspec((1, 8)),
            wspec((INNER, DIM)), wspec((1, DIM)),
            wspec((1, 8)),
        ],
        out_specs=[
            pl.BlockSpec((TQ, DIM), lambda i: (i, 0)),
            pl.BlockSpec((TQ, 16), lambda i: (i, 0)),
        ],
        out_shape=[
            jax.ShapeDtypeStruct((rows, DIM), jnp.float32),
            jax.ShapeDtypeStruct((rows, 16), jnp.float32),
        ],
    )(q, kv_nb, c_nb, coorsP, sv, pv, Wp1p, bp1, Wp2, bp2, W1a, W1b, be1,
      We2, be2, Wac, bac, Wac2, bac2, W_out, b_out, WcoP)


def _prep_consts(Wp1, bp1, Wp2, bp2, We1, be1, We2, be2,
                 Wa1, ba1, Wa2, ba2, Wc1, bc1, Wc2, bc2, Wco, W_out, b_out):
    f32 = jnp.float32
    scales = 2.0 ** jnp.arange(FF, dtype=f32)
    sv = jnp.zeros((1, 16), f32)
    sv = sv.at[0, :FF].set(1.0 / scales).at[0, FF:2 * FF].set(1.0 / scales)
    pv = jnp.zeros((1, 16), f32).at[0, FF:2 * FF].set(0.5 * math.pi)
    Wp1p = jnp.zeros((16, 2 * DH), f32).at[: FF * 2 + 1].set(Wp1)
    W1a, W1b = We1[:DH], We1[DH:]
    Wac1h = jnp.concatenate([Wa1, Wc1], axis=1)
    bac1h = jnp.concatenate([ba1, bc1], axis=0)
    Wac = jnp.zeros((H * M_D, H * 8 * M_D), f32)
    for h in range(H):
        Wac = Wac.at[h * M_D:(h + 1) * M_D,
                     h * 8 * M_D:(h + 1) * 8 * M_D].set(Wac1h)
    bac = jnp.tile(bac1h, (H,))[None]
    Wac2 = jnp.zeros((H * 8 * M_D, 8), f32)
    bac2 = jnp.zeros((1, 8), f32)
    for h in range(H):
        r0 = h * 8 * M_D
        Wac2 = Wac2.at[r0:r0 + 4 * M_D, 2 * h:2 * h + 1].set(Wa2)
        Wac2 = Wac2.at[r0 + 4 * M_D:r0 + 8 * M_D,
                       2 * h + 1:2 * h + 2].set(Wc2)
        bac2 = bac2.at[0, 2 * h].set(ba2[0]).at[0, 2 * h + 1].set(bc2[0])
    WcoP = jnp.zeros((1, 8), f32).at[0, :H].set(Wco[:, 0])
    return (sv, pv, Wp1p, bp1[None], Wp2, bp2[None], W1a, W1b, be1[None],
            We2, be2[None], Wac, bac, Wac2, bac2, W_out, b_out[None], WcoP)


def kernel(feats, coors, W_qkv, W_out, b_out, Wp1, bp1, Wp2, bp2,
           We1, be1, We2, be2, Wa1, ba1, Wa2, ba2, Wc1, bc1, Wc2, bc2, Wco):
    coorsP = jnp.pad(coors, ((0, 0), (0, 0), (0, 13)))
    coors_kT = jnp.swapaxes(coorsP, 1, 2)
    q, kv, d2 = _stage_a(feats, W_qkv, coorsP, coors_kT)
    consts = _prep_consts(Wp1, bp1, Wp2, bp2, We1, be1, We2, be2,
                          Wa1, ba1, Wa2, ba2, Wc1, bc1, Wc2, bc2, Wco,
                          W_out, b_out)
    nbs = [_sc_select_gather(d2[b], kv[b], coors_kT[b], N) for b in range(B)]
    outs, couts = [], []
    for b in range(B):
        kv_nb, c_nb = nbs[b]
        o, co = _stage_c(q[b], kv_nb, c_nb, coorsP[b], consts, N)
        outs.append(o)
        couts.append(co[:, :3])
    return jnp.stack(outs), jnp.stack(couts)
